# Initial kernel scaffold; baseline (speedup 1.0000x reference)
import jax
import jax.numpy as jnp
from jax.experimental import pallas as pl


def kernel(center, context, negative, input_table, output_table):
    raise NotImplementedError("write your pallas kernel here")


# speedup vs baseline: 3.9804x; 3.9804x over previous
import functools

import jax
import jax.numpy as jnp
from jax import lax
from jax.experimental import pallas as pl
from jax.experimental.pallas import tpu as pltpu
from jax.experimental.pallas import tpu_sc as plsc

_VOCAB = 1000000
_EMBED = 64
_BATCH = 16384
_NEG = 20
_COLS = _NEG + 1

_NC, _NS = 2, 16
_NW = _NC * _NS
_ROWS_PER_W = _BATCH // _NW
_R = 32
_NCHUNK = _ROWS_PER_W // _R
_CNROWS = _R * _COLS
_GPC = 6
_GLEN = _CNROWS // _GPC


def _sc_scores(center_idx, cn_idx, input_table, output_table):
    mesh = plsc.VectorSubcoreMesh(core_axis_name="c", subcore_axis_name="s")

    @functools.partial(
        pl.kernel,
        out_type=jax.ShapeDtypeStruct((_NW, _NCHUNK, _COLS, _R), jnp.float32),
        mesh=mesh,
        scratch_types=[
            pltpu.VMEM((_R,), jnp.int32),
            pltpu.VMEM((_GPC, _GLEN), jnp.int32),
            pltpu.VMEM((_R, _EMBED), jnp.float32),
            pltpu.VMEM((_CNROWS, _EMBED), jnp.float32),
            pltpu.VMEM((_COLS, _R), jnp.float32),
            pltpu.SemaphoreType.DMA,
        ],
        compiler_params=pltpu.CompilerParams(
            use_tc_tiling_on_sc=False, needs_layout_passes=False),
    )
    def scores_kernel(cidx_hbm, cnidx_hbm, itab_hbm, otab_hbm, out_hbm,
                      cidx_v, cnidx_v, crows_v, cnrows_v, scores_v, sem):
        wid = lax.axis_index("s") * _NC + lax.axis_index("c")
        lanes = lax.iota(jnp.int32, _NS)

        def chunk_body(c, _):
            pltpu.sync_copy(cidx_hbm.at[wid, c], cidx_v)
            pltpu.sync_copy(cnidx_hbm.at[wid, c], cnidx_v)
            pltpu.async_copy(itab_hbm.at[cidx_v], crows_v, sem).wait()
            copies = [
                pltpu.async_copy(
                    otab_hbm.at[cnidx_v.at[j]],
                    cnrows_v.at[pl.ds(j * _GLEN, _GLEN)], sem)
                for j in range(_GPC)
            ]
            for cp in copies:
                cp.wait()

            for g in range(_R // _NS):
                r_vec = g * _NS + lanes
                cn_rows = [r_vec * _COLS + k for k in range(_COLS)]

                def d_body(d, accs):
                    d_vec = jnp.full((_NS,), d, jnp.int32)
                    cen = plsc.load_gather(crows_v, [r_vec, d_vec])
                    return tuple(
                        accs[k] + cen * plsc.load_gather(
                            cnrows_v, [cn_rows[k], d_vec])
                        for k in range(_COLS)
                    )

                accs = lax.fori_loop(
                    0, _EMBED, d_body,
                    tuple(jnp.zeros((_NS,), jnp.float32)
                          for _ in range(_COLS)))
                scores_v[0, pl.ds(g * _NS, _NS)] = -accs[0]
                for k in range(1, _COLS):
                    scores_v[k, pl.ds(g * _NS, _NS)] = accs[k]

            pltpu.sync_copy(scores_v, out_hbm.at[wid, c])
            return _

        lax.fori_loop(0, _NCHUNK, chunk_body, 0)

    return scores_kernel(center_idx, cn_idx, input_table, output_table)


def _loss_body(s_ref, o_ref):
    t = s_ref[...]
    sp = jnp.maximum(t, 0.0) + jnp.log1p(jnp.exp(-jnp.abs(t)))
    o_ref[0, 0] = jnp.sum(sp) * (1.0 / _BATCH)


def kernel(center, context, negative, input_table, output_table):
    cn = jnp.concatenate([context[:, None], negative], axis=1)
    cn = cn.reshape(_NW, _NCHUNK, _GPC, _GLEN).astype(jnp.int32)
    cidx = center.reshape(_NW, _NCHUNK, _R).astype(jnp.int32)

    scores = _sc_scores(cidx, cn, input_table, output_table)

    flat = scores.reshape(_BATCH * _COLS // 128, 128)
    loss = pl.pallas_call(
        _loss_body,
        out_shape=jax.ShapeDtypeStruct((1, 1), jnp.float32),
        out_specs=pl.BlockSpec(memory_space=pltpu.SMEM),
    )(flat)
    return loss[0, 0]


# speedup vs baseline: 4.0982x; 1.0296x over previous
import functools

import jax
import jax.numpy as jnp
from jax import lax
from jax.experimental import pallas as pl
from jax.experimental.pallas import tpu as pltpu
from jax.experimental.pallas import tpu_sc as plsc

_VOCAB = 1000000
_EMBED = 64
_BATCH = 16384
_NEG = 20
_COLS = _NEG + 1

_NC, _NS = 2, 16
_NW = _NC * _NS
_ROWS_PER_W = _BATCH // _NW
_R = 32
_NCHUNK = _ROWS_PER_W // _R
_CNROWS = _R * _COLS
_GPC = 24
_GLEN = _CNROWS // _GPC
_CGP = 2
_CGL = _R // _CGP


def _sc_scores(center_idx, cn_idx, input_table, output_table):
    mesh = plsc.VectorSubcoreMesh(core_axis_name="c", subcore_axis_name="s")

    @functools.partial(
        pl.kernel,
        out_type=jax.ShapeDtypeStruct((_NW, _NCHUNK, _COLS, _R), jnp.float32),
        mesh=mesh,
        scratch_types=[
            pltpu.VMEM((_CGP, _CGL), jnp.int32),
            pltpu.VMEM((_CGP, _CGL), jnp.int32),
            pltpu.VMEM((_GPC, _GLEN), jnp.int32),
            pltpu.VMEM((_GPC, _GLEN), jnp.int32),
            pltpu.VMEM((_R, _EMBED), jnp.float32),
            pltpu.VMEM((_R, _EMBED), jnp.float32),
            pltpu.VMEM((_CNROWS, _EMBED), jnp.float32),
            pltpu.VMEM((_CNROWS, _EMBED), jnp.float32),
            pltpu.VMEM((_COLS, _R), jnp.float32),
            pltpu.VMEM((_COLS, _R), jnp.float32),
            pltpu.SemaphoreType.DMA,
            pltpu.SemaphoreType.DMA,
            pltpu.SemaphoreType.DMA,
            pltpu.SemaphoreType.DMA,
            pltpu.SemaphoreType.DMA,
            pltpu.SemaphoreType.DMA,
        ],
        compiler_params=pltpu.CompilerParams(
            use_tc_tiling_on_sc=False, needs_layout_passes=False),
    )
    def scores_kernel(cidx_hbm, cnidx_hbm, itab_hbm, otab_hbm, out_hbm,
                      cidx0, cidx1, cnidx0, cnidx1, crows0, crows1,
                      cnrows0, cnrows1, scores0, scores1,
                      isem0, isem1, gsem0, gsem1, osem0, osem1):
        wid = lax.axis_index("s") * _NC + lax.axis_index("c")
        lanes = lax.iota(jnp.int32, _NS)
        cidx = (cidx0, cidx1)
        cnidx = (cnidx0, cnidx1)
        crows = (crows0, crows1)
        cnrows = (cnrows0, cnrows1)
        scores = (scores0, scores1)
        isem = (isem0, isem1)
        gsem = (gsem0, gsem1)
        osem = (osem0, osem1)

        def idx_copies(c, p):
            return (
                pltpu.make_async_copy(cidx_hbm.at[wid, c], cidx[p], isem[p]),
                pltpu.make_async_copy(cnidx_hbm.at[wid, c], cnidx[p], isem[p]),
            )

        def fire_idx(c, p):
            for cp in idx_copies(c, p):
                cp.start()

        def drain_idx(c, p):
            for cp in idx_copies(c, p):
                cp.wait()

        def gather_copies(p):
            cps = [
                pltpu.make_async_copy(
                    itab_hbm.at[cidx[p].at[j]],
                    crows[p].at[pl.ds(j * _CGL, _CGL)], gsem[p])
                for j in range(_CGP)
            ]
            cps += [
                pltpu.make_async_copy(
                    otab_hbm.at[cnidx[p].at[j]],
                    cnrows[p].at[pl.ds(j * _GLEN, _GLEN)], gsem[p])
                for j in range(_GPC)
            ]
            return cps

        def fire_gathers(p):
            for cp in gather_copies(p):
                cp.start()

        def drain_gathers(p):
            for cp in gather_copies(p):
                cp.wait()

        def out_copy(c, p):
            return pltpu.make_async_copy(scores[p], out_hbm.at[wid, c],
                                         osem[p])

        def compute(p):
            for g in range(_R // _NS):
                r_vec = g * _NS + lanes
                cn_rows = [r_vec * _COLS + k for k in range(_COLS)]

                def d_body(d, accs):
                    d_vec = jnp.full((_NS,), d, jnp.int32)
                    cen = plsc.load_gather(crows[p], [r_vec, d_vec])
                    return tuple(
                        accs[k] + cen * plsc.load_gather(
                            cnrows[p], [cn_rows[k], d_vec])
                        for k in range(_COLS)
                    )

                accs = lax.fori_loop(
                    0, _EMBED, d_body,
                    tuple(jnp.zeros((_NS,), jnp.float32)
                          for _ in range(_COLS)))
                scores[p][0, pl.ds(g * _NS, _NS)] = -accs[0]
                for k in range(1, _COLS):
                    scores[p][k, pl.ds(g * _NS, _NS)] = accs[k]

        fire_idx(0, 0)
        drain_idx(0, 0)
        fire_gathers(0)
        fire_idx(1, 1)
        out_copy(0, 0).start()
        out_copy(1, 1).start()

        def pair_body(i, carry):
            c0 = 2 * i
            c1 = c0 + 1
            c2 = (c0 + 2) & (_NCHUNK - 1)
            c3 = (c0 + 3) & (_NCHUNK - 1)

            drain_idx(c1, 1)
            fire_gathers(1)
            drain_gathers(0)
            fire_idx(c2, 0)
            out_copy(c0, 0).wait()
            compute(0)
            out_copy(c0, 0).start()

            drain_idx(c2, 0)
            fire_gathers(0)
            drain_gathers(1)
            fire_idx(c3, 1)
            out_copy(c1, 1).wait()
            compute(1)
            out_copy(c1, 1).start()
            return carry

        lax.fori_loop(0, _NCHUNK // 2, pair_body, 0)

        drain_idx(1, 1)
        drain_gathers(0)
        out_copy(_NCHUNK - 2, 0).wait()
        out_copy(_NCHUNK - 1, 1).wait()

    return scores_kernel(center_idx, cn_idx, input_table, output_table)


def _loss_body(s_ref, o_ref):
    t = s_ref[...]
    sp = jnp.maximum(t, 0.0) + jnp.log1p(jnp.exp(-jnp.abs(t)))
    o_ref[0, 0] = jnp.sum(sp) * (1.0 / _BATCH)


def kernel(center, context, negative, input_table, output_table):
    cn = jnp.concatenate([context[:, None], negative], axis=1)
    cn = cn.reshape(_NW, _NCHUNK, _GPC, _GLEN).astype(jnp.int32)
    cidx = center.reshape(_NW, _NCHUNK, _CGP, _CGL).astype(jnp.int32)

    scores = _sc_scores(cidx, cn, input_table, output_table)

    flat = scores.reshape(_BATCH * _COLS // 128, 128)
    loss = pl.pallas_call(
        _loss_body,
        out_shape=jax.ShapeDtypeStruct((1, 1), jnp.float32),
        out_specs=pl.BlockSpec(memory_space=pltpu.SMEM),
    )(flat)
    return loss[0, 0]


# speedup vs baseline: 4.1031x; 1.0012x over previous
import functools

import jax
import jax.numpy as jnp
from jax import lax
from jax.experimental import pallas as pl
from jax.experimental.pallas import tpu as pltpu
from jax.experimental.pallas import tpu_sc as plsc

_VOCAB = 1000000
_EMBED = 64
_BATCH = 16384
_NEG = 20
_COLS = _NEG + 1

_NC, _NS = 2, 16
_NW = _NC * _NS
_ROWS_PER_W = _BATCH // _NW
_R = 32
_NCHUNK = _ROWS_PER_W // _R
_CNROWS = _R * _COLS
_GLEN = 16
_GPC = _CNROWS // _GLEN
_CGP = 2
_CGL = _R // _CGP


def _sc_scores(center_idx, cn_idx, input_table, output_table):
    mesh = plsc.VectorSubcoreMesh(core_axis_name="c", subcore_axis_name="s")

    @functools.partial(
        pl.kernel,
        out_type=jax.ShapeDtypeStruct((_NW, _NCHUNK, _COLS, _R), jnp.float32),
        mesh=mesh,
        scratch_types=[
            pltpu.VMEM((_CGP, _CGL), jnp.int32),
            pltpu.VMEM((_CGP, _CGL), jnp.int32),
            pltpu.VMEM((_GPC, _GLEN), jnp.int32),
            pltpu.VMEM((_GPC, _GLEN), jnp.int32),
            pltpu.VMEM((_R, _EMBED), jnp.float32),
            pltpu.VMEM((_R, _EMBED), jnp.float32),
            pltpu.VMEM((_CNROWS, _EMBED), jnp.float32),
            pltpu.VMEM((_CNROWS, _EMBED), jnp.float32),
            pltpu.VMEM((_COLS, _R), jnp.float32),
            pltpu.VMEM((_COLS, _R), jnp.float32),
            pltpu.SemaphoreType.DMA,
            pltpu.SemaphoreType.DMA,
            pltpu.SemaphoreType.DMA,
            pltpu.SemaphoreType.DMA,
            pltpu.SemaphoreType.DMA,
            pltpu.SemaphoreType.DMA,
        ],
        compiler_params=pltpu.CompilerParams(
            use_tc_tiling_on_sc=False, needs_layout_passes=False),
    )
    def scores_kernel(cidx_hbm, cnidx_hbm, itab_hbm, otab_hbm, out_hbm,
                      cidx0, cidx1, cnidx0, cnidx1, crows0, crows1,
                      cnrows0, cnrows1, scores0, scores1,
                      isem0, isem1, gsem0, gsem1, osem0, osem1):
        wid = lax.axis_index("s") * _NC + lax.axis_index("c")
        lanes = lax.iota(jnp.int32, _NS)
        cidx = (cidx0, cidx1)
        cnidx = (cnidx0, cnidx1)
        crows = (crows0, crows1)
        cnrows = (cnrows0, cnrows1)
        scores = (scores0, scores1)
        isem = (isem0, isem1)
        gsem = (gsem0, gsem1)
        osem = (osem0, osem1)

        def idx_copies(c, p):
            return (
                pltpu.make_async_copy(cidx_hbm.at[wid, c], cidx[p], isem[p]),
                pltpu.make_async_copy(cnidx_hbm.at[wid, c], cnidx[p], isem[p]),
            )

        def fire_idx(c, p):
            for cp in idx_copies(c, p):
                cp.start()

        def drain_idx(c, p):
            for cp in idx_copies(c, p):
                cp.wait()

        def fire_gathers(p):
            for j in range(_CGP):
                pltpu.make_async_copy(
                    itab_hbm.at[cidx[p][j, :]],
                    crows[p].at[pl.ds(j * _CGL, _CGL)], gsem[p]).start()
            for j in range(_GPC):
                pltpu.make_async_copy(
                    otab_hbm.at[cnidx[p][j, :]],
                    cnrows[p].at[pl.ds(j * _GLEN, _GLEN)], gsem[p]).start()

        def drain_gathers(p):
            pltpu.make_async_copy(
                itab_hbm.at[pl.ds(0, _R)], crows[p], gsem[p]).wait()
            pltpu.make_async_copy(
                otab_hbm.at[pl.ds(0, _CNROWS)], cnrows[p], gsem[p]).wait()

        def out_copy(c, p):
            return pltpu.make_async_copy(scores[p], out_hbm.at[wid, c],
                                         osem[p])

        def compute(p):
            for g in range(_R // _NS):
                r_vec = g * _NS + lanes
                cn_rows = [r_vec * _COLS + k for k in range(_COLS)]

                def d_body(d, accs):
                    d_vec = jnp.full((_NS,), d, jnp.int32)
                    cen = plsc.load_gather(crows[p], [r_vec, d_vec])
                    return tuple(
                        accs[k] + cen * plsc.load_gather(
                            cnrows[p], [cn_rows[k], d_vec])
                        for k in range(_COLS)
                    )

                accs = lax.fori_loop(
                    0, _EMBED, d_body,
                    tuple(jnp.zeros((_NS,), jnp.float32)
                          for _ in range(_COLS)))
                scores[p][0, pl.ds(g * _NS, _NS)] = -accs[0]
                for k in range(1, _COLS):
                    scores[p][k, pl.ds(g * _NS, _NS)] = accs[k]

        fire_idx(0, 0)
        drain_idx(0, 0)
        fire_gathers(0)
        fire_idx(1, 1)
        out_copy(0, 0).start()
        out_copy(1, 1).start()

        def pair_body(i, carry):
            c0 = 2 * i
            c1 = c0 + 1
            c2 = (c0 + 2) & (_NCHUNK - 1)
            c3 = (c0 + 3) & (_NCHUNK - 1)

            drain_idx(c1, 1)
            fire_gathers(1)
            drain_gathers(0)
            fire_idx(c2, 0)
            out_copy(c0, 0).wait()
            compute(0)
            out_copy(c0, 0).start()

            drain_idx(c2, 0)
            fire_gathers(0)
            drain_gathers(1)
            fire_idx(c3, 1)
            out_copy(c1, 1).wait()
            compute(1)
            out_copy(c1, 1).start()
            return carry

        lax.fori_loop(0, _NCHUNK // 2, pair_body, 0)

        drain_idx(1, 1)
        drain_gathers(0)
        out_copy(_NCHUNK - 2, 0).wait()
        out_copy(_NCHUNK - 1, 1).wait()

    return scores_kernel(center_idx, cn_idx, input_table, output_table)


def _loss_body(s_ref, o_ref):
    t = s_ref[...]
    sp = jnp.maximum(t, 0.0) + jnp.log1p(jnp.exp(-jnp.abs(t)))
    o_ref[0, 0] = jnp.sum(sp) * (1.0 / _BATCH)


def kernel(center, context, negative, input_table, output_table):
    cn = jnp.concatenate([context[:, None], negative], axis=1)
    cn = cn.reshape(_NW, _NCHUNK, _GPC, _GLEN).astype(jnp.int32)
    cidx = center.reshape(_NW, _NCHUNK, _CGP, _CGL).astype(jnp.int32)

    scores = _sc_scores(cidx, cn, input_table, output_table)

    flat = scores.reshape(_BATCH * _COLS // 128, 128)
    loss = pl.pallas_call(
        _loss_body,
        out_shape=jax.ShapeDtypeStruct((1, 1), jnp.float32),
        out_specs=pl.BlockSpec(memory_space=pltpu.SMEM),
    )(flat)
    return loss[0, 0]
